# Initial kernel scaffold; baseline (speedup 1.0000x reference)
import functools

import jax
import jax.numpy as jnp
from jax import lax
from jax.experimental import pallas as pl
from jax.experimental.pallas import tpu as pltpu
from jax.experimental.pallas import tpu_sc as plsc

B = 256
N_IN = 16384
N_OUT = 16384
FAN = 16
NNZ = N_OUT * FAN
NC = 2
NS = 16
NW = NC * NS
RPW = N_OUT // NW
BLK = 8
NBLK = RPW // BLK
IDX_PER_BLK = BLK * FAN
L = 16

_LANE_ZERO = jnp.zeros((L,), jnp.int32)


def _body(xT_hbm, cols2_hbm, w_hbm, bias_hbm, outT_hbm,
          idx_all, w_all, bias_w, gath, out_buf, gsem):
    cid = lax.axis_index("c")
    sid = lax.axis_index("s")
    wid = sid * NC + cid
    base_row = wid * RPW
    base_conn = wid * (RPW * FAN)

    pltpu.sync_copy(cols2_hbm.at[pl.ds(wid * NBLK, NBLK)], idx_all)
    pltpu.sync_copy(w_hbm.at[pl.ds(base_conn, RPW * FAN)], w_all)
    pltpu.sync_copy(bias_hbm.at[pl.ds(base_row, RPW)], bias_w)

    def block(blk, carry):
        pltpu.async_copy(xT_hbm.at[idx_all.at[blk]], gath, gsem).wait()
        for j in range(BLK):
            r = blk * BLK + j
            bias_b = plsc.load_gather(bias_w, [_LANE_ZERO + r])
            def chunk(c16, carry2):
                off = c16 * L
                acc = bias_b
                for f in range(FAN):
                    wb = plsc.load_gather(w_all, [_LANE_ZERO + (r * FAN + f)])
                    acc = acc + gath[j * FAN + f, pl.ds(off, L)] * wb
                out_buf[j, pl.ds(off, L)] = acc
                return carry2
            lax.fori_loop(0, B // L, chunk, 0)
        pltpu.sync_copy(out_buf, outT_hbm.at[pl.ds(base_row + blk * BLK, BLK)])
        return carry

    lax.fori_loop(0, NBLK, block, 0)


@functools.partial(
    pl.kernel,
    out_type=jax.ShapeDtypeStruct((N_OUT, B), jnp.float32),
    mesh=plsc.VectorSubcoreMesh(core_axis_name="c", subcore_axis_name="s",
                                num_cores=NC, num_subcores=NS),
    scratch_types=[
        pltpu.VMEM((NBLK, IDX_PER_BLK), jnp.int32),
        pltpu.VMEM((RPW * FAN,), jnp.float32),
        pltpu.VMEM((RPW,), jnp.float32),
        pltpu.VMEM((IDX_PER_BLK, B), jnp.float32),
        pltpu.VMEM((BLK, B), jnp.float32),
        pltpu.SemaphoreType.DMA,
    ],
)
def _local_linear_sc(xT_hbm, cols2_hbm, w_hbm, bias_hbm, outT_hbm,
                     idx_all, w_all, bias_w, gath, out_buf, gsem):
    _body(xT_hbm, cols2_hbm, w_hbm, bias_hbm, outT_hbm,
          idx_all, w_all, bias_w, gath, out_buf, gsem)


def kernel(x, rows, cols, weight, bias):
    del rows
    xT = x.T
    cols2 = cols.astype(jnp.int32).reshape(NW * NBLK, IDX_PER_BLK)
    outT = _local_linear_sc(xT, cols2, weight.astype(jnp.float32),
                            bias.astype(jnp.float32))
    return outT.T


# speedup vs baseline: 7.1300x; 7.1300x over previous
import functools

import jax
import jax.numpy as jnp
from jax import lax
from jax.experimental import pallas as pl
from jax.experimental.pallas import tpu as pltpu
from jax.experimental.pallas import tpu_sc as plsc

B = 256
N_IN = 16384
N_OUT = 16384
FAN = 16
NNZ = N_OUT * FAN
NC = 2
NS = 16
NW = NC * NS
RPW = N_OUT // NW
BLK = 8
NBLK = RPW // BLK
IDX_PER_BLK = BLK * FAN
L = 16

def _dyn_gather(vec, idx):
    dnums = lax.GatherDimensionNumbers(
        offset_dims=(), collapsed_slice_dims=(0,), start_index_map=(0,))
    return lax.gather(vec, idx[:, None], dnums, slice_sizes=(1,),
                      mode=lax.GatherScatterMode.PROMISE_IN_BOUNDS)


def _body(xT_hbm, cols_hbm, w_hbm, biasx_hbm, outT_hbm,
          idx_buf, w_all, bias_w, gath, out_buf, gsem):
    lane_f = [jnp.full((L,), f, jnp.int32) for f in range(FAN)]
    cid = lax.axis_index("c")
    sid = lax.axis_index("s")
    wid = sid * NC + cid
    base_row = wid * RPW
    base_conn = wid * (RPW * FAN)

    pltpu.sync_copy(w_hbm.at[pl.ds(base_conn, RPW * FAN)], w_all)
    pltpu.sync_copy(biasx_hbm.at[pl.ds(base_row * L, RPW * L)], bias_w)

    def block(blk, carry):
        pltpu.sync_copy(cols_hbm.at[pl.ds(base_conn + blk * IDX_PER_BLK,
                                          IDX_PER_BLK)], idx_buf)
        pltpu.async_copy(xT_hbm.at[idx_buf], gath, gsem).wait()
        for j in range(BLK):
            r = blk * BLK + j
            wrow = w_all[pl.ds(r * FAN, FAN)]
            bias_b = bias_w[pl.ds(r * L, L)]
            wb = [_dyn_gather(wrow, lane_f[f]) for f in range(FAN)]
            def chunk(c16, carry2):
                off = c16 * L
                acc = bias_b
                for f in range(FAN):
                    acc = acc + gath[j * FAN + f, pl.ds(off, L)] * wb[f]
                out_buf[j, pl.ds(off, L)] = acc
                return carry2
            lax.fori_loop(0, B // L, chunk, 0)
        pltpu.sync_copy(out_buf, outT_hbm.at[pl.ds(base_row + blk * BLK, BLK)])
        return carry

    lax.fori_loop(0, NBLK, block, 0)


@functools.partial(
    pl.kernel,
    out_type=jax.ShapeDtypeStruct((N_OUT, B), jnp.float32),
    mesh=plsc.VectorSubcoreMesh(core_axis_name="c", subcore_axis_name="s",
                                num_cores=NC, num_subcores=NS),
    scratch_types=[
        pltpu.VMEM((IDX_PER_BLK,), jnp.int32),
        pltpu.VMEM((RPW * FAN,), jnp.float32),
        pltpu.VMEM((RPW * L,), jnp.float32),
        pltpu.VMEM((IDX_PER_BLK, B), jnp.float32),
        pltpu.VMEM((BLK, B), jnp.float32),
        pltpu.SemaphoreType.DMA,
    ],
)
def _local_linear_sc(xT_hbm, cols_hbm, w_hbm, biasx_hbm, outT_hbm,
                     idx_buf, w_all, bias_w, gath, out_buf, gsem):
    _body(xT_hbm, cols_hbm, w_hbm, biasx_hbm, outT_hbm,
          idx_buf, w_all, bias_w, gath, out_buf, gsem)


def kernel(x, rows, cols, weight, bias):
    del rows
    xT = x.T
    cols_i = cols.astype(jnp.int32)
    biasx = jnp.broadcast_to(bias.astype(jnp.float32)[:, None],
                             (N_OUT, L)).reshape(N_OUT * L)
    outT = _local_linear_sc(xT, cols_i, weight.astype(jnp.float32), biasx)
    return outT.T


# speedup vs baseline: 11.8285x; 1.6590x over previous
import functools

import jax
import jax.numpy as jnp
from jax import lax
from jax.experimental import pallas as pl
from jax.experimental.pallas import tpu as pltpu
from jax.experimental.pallas import tpu_sc as plsc

B = 256
N_IN = 16384
N_OUT = 16384
FAN = 16
NNZ = N_OUT * FAN
NC = 2
NS = 16
NW = NC * NS
RPW = N_OUT // NW
BLK = 8
NBLK = RPW // BLK
IDX_PER_BLK = BLK * FAN
L = 16

def _dyn_gather(vec, idx):
    dnums = lax.GatherDimensionNumbers(
        offset_dims=(), collapsed_slice_dims=(0,), start_index_map=(0,))
    return lax.gather(vec, idx[:, None], dnums, slice_sizes=(1,),
                      mode=lax.GatherScatterMode.PROMISE_IN_BOUNDS)


def _body(xT_hbm, cols_hbm, w_hbm, biasx_hbm, outT_hbm,
          idx0, idx1, w_all, bias_w, gath0, gath1, out0, out1,
          gsem0, gsem1, isem0, isem1, osem0, osem1):
    lane_f = [jnp.full((L,), f, jnp.int32) for f in range(FAN)]
    cid = lax.axis_index("c")
    sid = lax.axis_index("s")
    wid = sid * NC + cid
    base_row = wid * RPW
    base_conn = wid * (RPW * FAN)

    pltpu.sync_copy(w_hbm.at[pl.ds(base_conn, RPW * FAN)], w_all)
    pltpu.sync_copy(biasx_hbm.at[pl.ds(base_row * L, RPW * L)], bias_w)

    def idx_src(blk):
        return cols_hbm.at[pl.ds(base_conn + blk * IDX_PER_BLK, IDX_PER_BLK)]

    def compute(gath, out_buf, blk):
        for j in range(BLK):
            r = blk * BLK + j
            wrow = w_all[pl.ds(r * FAN, FAN)]
            bias_b = bias_w[pl.ds(r * L, L)]
            wb = [_dyn_gather(wrow, lane_f[f]) for f in range(FAN)]
            def chunk(c16, carry2):
                off = c16 * L
                acc = bias_b
                for f in range(FAN):
                    acc = acc + gath[j * FAN + f, pl.ds(off, L)] * wb[f]
                out_buf[j, pl.ds(off, L)] = acc
                return carry2
            lax.fori_loop(0, B // L, chunk, 0)

    def out_dst(blk):
        return outT_hbm.at[pl.ds(base_row + blk * BLK, BLK)]

    pltpu.sync_copy(idx_src(0), idx0)
    pltpu.sync_copy(idx_src(1), idx1)
    pltpu.async_copy(xT_hbm.at[idx0], gath0, gsem0)

    def pair(t, carry):
        b0 = 2 * t
        not_last = t < NBLK // 2 - 1
        not_first = t > 0
        pltpu.async_copy(xT_hbm.at[idx1], gath1, gsem1)
        pltpu.make_async_copy(xT_hbm.at[idx0], gath0, gsem0).wait()
        @pl.when(not_last)
        def _():
            pltpu.async_copy(idx_src(b0 + 2), idx0, isem0)
        @pl.when(not_first)
        def _():
            pltpu.make_async_copy(out0, out_dst(b0 - 2), osem0).wait()
        compute(gath0, out0, b0)
        pltpu.async_copy(out0, out_dst(b0), osem0)
        @pl.when(not_last)
        def _():
            pltpu.make_async_copy(idx_src(b0 + 2), idx0, isem0).wait()
            pltpu.async_copy(xT_hbm.at[idx0], gath0, gsem0)
        pltpu.make_async_copy(xT_hbm.at[idx1], gath1, gsem1).wait()
        @pl.when(not_last)
        def _():
            pltpu.async_copy(idx_src(b0 + 3), idx1, isem1)
        @pl.when(not_first)
        def _():
            pltpu.make_async_copy(out1, out_dst(b0 - 1), osem1).wait()
        compute(gath1, out1, b0 + 1)
        pltpu.async_copy(out1, out_dst(b0 + 1), osem1)
        @pl.when(not_last)
        def _():
            pltpu.make_async_copy(idx_src(b0 + 3), idx1, isem1).wait()
        return carry

    lax.fori_loop(0, NBLK // 2, pair, 0)
    pltpu.make_async_copy(out0, out_dst(NBLK - 2), osem0).wait()
    pltpu.make_async_copy(out1, out_dst(NBLK - 1), osem1).wait()


@functools.partial(
    pl.kernel,
    out_type=jax.ShapeDtypeStruct((N_OUT, B), jnp.float32),
    mesh=plsc.VectorSubcoreMesh(core_axis_name="c", subcore_axis_name="s",
                                num_cores=NC, num_subcores=NS),
    scratch_types=[
        pltpu.VMEM((IDX_PER_BLK,), jnp.int32),
        pltpu.VMEM((IDX_PER_BLK,), jnp.int32),
        pltpu.VMEM((RPW * FAN,), jnp.float32),
        pltpu.VMEM((RPW * L,), jnp.float32),
        pltpu.VMEM((IDX_PER_BLK, B), jnp.float32),
        pltpu.VMEM((IDX_PER_BLK, B), jnp.float32),
        pltpu.VMEM((BLK, B), jnp.float32),
        pltpu.VMEM((BLK, B), jnp.float32),
        pltpu.SemaphoreType.DMA,
        pltpu.SemaphoreType.DMA,
        pltpu.SemaphoreType.DMA,
        pltpu.SemaphoreType.DMA,
        pltpu.SemaphoreType.DMA,
        pltpu.SemaphoreType.DMA,
    ],
)
def _local_linear_sc(xT_hbm, cols_hbm, w_hbm, biasx_hbm, outT_hbm,
                     idx0, idx1, w_all, bias_w, gath0, gath1, out0, out1,
                     gsem0, gsem1, isem0, isem1, osem0, osem1):
    _body(xT_hbm, cols_hbm, w_hbm, biasx_hbm, outT_hbm,
          idx0, idx1, w_all, bias_w, gath0, gath1, out0, out1,
          gsem0, gsem1, isem0, isem1, osem0, osem1)


def kernel(x, rows, cols, weight, bias):
    del rows
    xT = x.T
    cols_i = cols.astype(jnp.int32)
    biasx = jnp.broadcast_to(bias.astype(jnp.float32)[:, None],
                             (N_OUT, L)).reshape(N_OUT * L)
    outT = _local_linear_sc(xT, cols_i, weight.astype(jnp.float32), biasx)
    return outT.T
